# Initial kernel scaffold; baseline (speedup 1.0000x reference)
import jax
import jax.numpy as jnp
from jax.experimental import pallas as pl


def kernel(embeddings, positive_nodes, negative_nodes):
    raise NotImplementedError("write your pallas kernel here")


# speedup vs baseline: 1.4911x; 1.4911x over previous
import jax
import jax.numpy as jnp
from jax import lax
from jax.experimental import pallas as pl
from jax.experimental.pallas import tpu as pltpu
from jax.experimental.pallas import tpu_sc as plsc

_N = 8192
_D = 256
_NC = 2
_NS = 16
_NW = _NC * _NS
_RPW = _N // _NW
_CHUNK = 128
_NCHUNK = _RPW // _CHUNK
_G = _D // 16


def _sc_body(emb, pos, neg, s_out, q_out, idx_v, buf0, buf1, part_s, part_q,
             sem0, sem1):
    wid = lax.axis_index("s") * _NC + lax.axis_index("c")

    pltpu.sync_copy(pos.at[pl.ds(wid * _NCHUNK, _NCHUNK)],
                    idx_v.at[pl.ds(0, _NCHUNK)])
    pltpu.sync_copy(neg.at[pl.ds(wid * _NCHUNK, _NCHUNK)],
                    idx_v.at[pl.ds(_NCHUNK, _NCHUNK)])

    bufs = (buf0, buf1)
    sems = (sem0, sem1)
    ntot = 2 * _NCHUNK
    handles = [None, None]

    handles[0] = pltpu.async_copy(emb.at[idx_v.at[0]], bufs[0], sems[0])

    zero = jnp.zeros((16,), jnp.float32)
    accs = [[zero] * (_G + 1), [zero] * (_G + 1)]

    for j in range(ntot):
        if j + 1 < ntot:
            handles[(j + 1) % 2] = pltpu.async_copy(
                emb.at[idx_v.at[j + 1]], bufs[(j + 1) % 2], sems[(j + 1) % 2])
        handles[j % 2].wait()
        rbuf = bufs[j % 2]
        side = j // _NCHUNK

        def row_body(r, carry, rbuf=rbuf):
            out = []
            q = carry[_G]
            for g in range(_G):
                v = rbuf[r, pl.ds(g * 16, 16)]
                out.append(carry[g] + v)
                q = q + v * v
            out.append(q)
            return tuple(out)

        accs[side] = list(lax.fori_loop(0, _CHUNK, row_body, tuple(accs[side])))

    for side in range(2):
        for g in range(_G):
            part_s[side, pl.ds(g * 16, 16)] = accs[side][g]
        part_q[side, :] = accs[side][_G]

    pltpu.sync_copy(part_s, s_out.at[wid])
    pltpu.sync_copy(part_q, q_out.at[wid])


def _fin_body(s_ref, q_ref, o_ref):
    s = jnp.sum(s_ref[...], axis=0)
    q = jnp.sum(q_ref[...])
    ss = jnp.sum(s * s)
    inv = 1.0 / (_N * _D)
    o_ref[...] = jnp.full((1, 1), q * inv - ss * inv / _N, jnp.float32)


def kernel(embeddings, positive_nodes, negative_nodes):
    pos = positive_nodes.astype(jnp.int32).reshape(_NW * _NCHUNK, _CHUNK)
    neg = negative_nodes.astype(jnp.int32).reshape(_NW * _NCHUNK, _CHUNK)
    mesh = plsc.VectorSubcoreMesh(core_axis_name="c", subcore_axis_name="s")
    sc = pl.kernel(
        _sc_body,
        out_type=(jax.ShapeDtypeStruct((_NW, 2, _D), jnp.float32),
                  jax.ShapeDtypeStruct((_NW, 2, 16), jnp.float32)),
        mesh=mesh,
        scratch_types=(
            pltpu.VMEM((2 * _NCHUNK, _CHUNK), jnp.int32),
            pltpu.VMEM((_CHUNK, _D), jnp.float32),
            pltpu.VMEM((_CHUNK, _D), jnp.float32),
            pltpu.VMEM((2, _D), jnp.float32),
            pltpu.VMEM((2, 16), jnp.float32),
            pltpu.SemaphoreType.DMA,
            pltpu.SemaphoreType.DMA,
        ),
    )
    s_parts, q_parts = sc(embeddings, pos, neg)
    out = pl.pallas_call(
        _fin_body,
        out_shape=jax.ShapeDtypeStruct((1, 1), jnp.float32),
    )(s_parts, q_parts)
    return out[0, 0]


# speedup vs baseline: 1.6483x; 1.1054x over previous
import jax
import jax.numpy as jnp
from jax import lax
from jax.experimental import pallas as pl
from jax.experimental.pallas import tpu as pltpu
from jax.experimental.pallas import tpu_sc as plsc

_N = 8192
_D = 256
_NC = 2
_NS = 16
_NW = _NC * _NS
_RPW = _N // _NW
_CHUNK = 128
_NCHUNK = _RPW // _CHUNK
_G = _D // 16


def _sc_body(emb, pos, neg, s_out, q_out, idx_v, buf0, buf1, part_s, part_q,
             sem0, sem1):
    wid = lax.axis_index("s") * _NC + lax.axis_index("c")

    pltpu.sync_copy(pos.at[pl.ds(wid * _NCHUNK, _NCHUNK)],
                    idx_v.at[pl.ds(0, _NCHUNK)])
    pltpu.sync_copy(neg.at[pl.ds(wid * _NCHUNK, _NCHUNK)],
                    idx_v.at[pl.ds(_NCHUNK, _NCHUNK)])

    bufs = (buf0, buf1)
    sems = (sem0, sem1)
    ntot = 2 * _NCHUNK
    handles = [None, None]

    handles[0] = pltpu.async_copy(emb.at[idx_v.at[0]], bufs[0], sems[0])

    zero = jnp.zeros((16,), jnp.float32)
    _NQ = 8
    accs = [[zero] * (_G + _NQ), [zero] * (_G + _NQ)]

    for j in range(ntot):
        if j + 1 < ntot:
            handles[(j + 1) % 2] = pltpu.async_copy(
                emb.at[idx_v.at[j + 1]], bufs[(j + 1) % 2], sems[(j + 1) % 2])
        handles[j % 2].wait()
        rbuf = bufs[j % 2]
        side = j // _NCHUNK

        def row_body(r, carry, rbuf=rbuf):
            s = list(carry[:_G])
            q = list(carry[_G:])
            for u in range(2):
                for g in range(_G):
                    v = rbuf[2 * r + u, pl.ds(g * 16, 16)]
                    s[g] = s[g] + v
                    q[(g + u * _NQ // 2) % _NQ] = q[(g + u * _NQ // 2) % _NQ] + v * v
            return tuple(s + q)

        accs[side] = list(
            lax.fori_loop(0, _CHUNK // 2, row_body, tuple(accs[side])))

    for side in range(2):
        for g in range(_G):
            part_s[side, pl.ds(g * 16, 16)] = accs[side][g]
        qtot = accs[side][_G]
        for u in range(1, _NQ):
            qtot = qtot + accs[side][_G + u]
        part_q[side, :] = qtot

    pltpu.sync_copy(part_s, s_out.at[wid])
    pltpu.sync_copy(part_q, q_out.at[wid])


def _fin_body(s_ref, q_ref, o_ref):
    s = jnp.sum(s_ref[...], axis=0)
    q = jnp.sum(q_ref[...])
    ss = jnp.sum(s * s)
    inv = 1.0 / (_N * _D)
    o_ref[...] = jnp.full((1, 1), q * inv - ss * inv / _N, jnp.float32)


def kernel(embeddings, positive_nodes, negative_nodes):
    pos = positive_nodes.astype(jnp.int32).reshape(_NW * _NCHUNK, _CHUNK)
    neg = negative_nodes.astype(jnp.int32).reshape(_NW * _NCHUNK, _CHUNK)
    mesh = plsc.VectorSubcoreMesh(core_axis_name="c", subcore_axis_name="s")
    sc = pl.kernel(
        _sc_body,
        out_type=(jax.ShapeDtypeStruct((_NW, 2, _D), jnp.float32),
                  jax.ShapeDtypeStruct((_NW, 2, 16), jnp.float32)),
        mesh=mesh,
        scratch_types=(
            pltpu.VMEM((2 * _NCHUNK, _CHUNK), jnp.int32),
            pltpu.VMEM((_CHUNK, _D), jnp.float32),
            pltpu.VMEM((_CHUNK, _D), jnp.float32),
            pltpu.VMEM((2, _D), jnp.float32),
            pltpu.VMEM((2, 16), jnp.float32),
            pltpu.SemaphoreType.DMA,
            pltpu.SemaphoreType.DMA,
        ),
    )
    s_parts, q_parts = sc(embeddings, pos, neg)
    out = pl.pallas_call(
        _fin_body,
        out_shape=jax.ShapeDtypeStruct((1, 1), jnp.float32),
    )(s_parts, q_parts)
    return out[0, 0]
